# baseline (speedup 1.0000x reference)
import jax
import jax.numpy as jnp
from jax import lax
from jax.experimental import pallas as pl
from jax.experimental.pallas import tpu as pltpu

B = 8
L = 2048
N = 8192
H = 256
EDIM = 256
NE = 512
BETA = 0.25

TB = 256
HB = 256
PBB = L // TB
NBLK = (B * L) // TB


def _rowsq_xla_tree(x):
    z2 = x * x
    b = z2[:, :128] + z2[:, 128:]
    t = b[:, 0:8]
    for j in range(1, 16):
        t = t + b[:, 8 * j:8 * j + 8]
    u1 = t[:, 0:4] + t[:, 4:8]
    u2 = u1[:, 0:2] + u1[:, 2:4]
    return u2[:, 0:1] + u2[:, 1:2]


def _rowsq_xla_row(x):
    z2 = x * x
    b = z2[:, :128] + z2[:, 128:]
    chunks = []
    for t in range(x.shape[0] // 128):
        bt = b[128 * t:128 * (t + 1), :].T
        s = bt[0:8, :]
        for j in range(1, 16):
            s = s + bt[8 * j:8 * (j + 1), :]
        s = s + pltpu.roll(s, 4, axis=0)
        s = s + pltpu.roll(s, 2, axis=0)
        s = s + pltpu.roll(s, 1, axis=0)
        chunks.append(s[0:1, :])
    return chunks[0] if len(chunks) == 1 else jnp.concatenate(chunks, axis=1)


def _fused(starts_ref, counts_ref, coords_ref,
           We0_ref, be0_ref, We1_ref, be1_ref, We2_ref, be2_ref,
           We3_ref, be3_ref, We4_ref, be4_ref, cb_ref,
           Wd0_ref, bd0_ref, Wd1_ref, bd1_ref, Wd2_ref, bd2_ref,
           enc_out, zq_out, rec_out, idx_out, mask_out, loss_out, perp_out,
           acc_sq, acc_cnt, acc_ecnt, pad_enc, pad_zq, pad_rec, pad_idx,
           cn_s):
    i = pl.program_id(0)
    b = i // PBB
    p0 = (i % PBB) * TB
    start = starts_ref[b]
    cnt = counts_ref[b]
    cb = cb_ref[...]

    def encode(x):
        h = jnp.maximum(x @ We0_ref[...] + be0_ref[...][None, :], 0.0)
        h = jnp.maximum(h @ We1_ref[...] + be1_ref[...][None, :], 0.0)
        h = jnp.maximum(h @ We2_ref[...] + be2_ref[...][None, :], 0.0)
        h = jnp.maximum(h @ We3_ref[...] + be3_ref[...][None, :], 0.0)
        enc = h @ We4_ref[...] + be4_ref[...][None, :]
        return jnp.clip(enc, -10.0, 10.0)

    def vq(enc, zn):
        d = zn + cn_s[...] - 2.0 * (enc @ cb.T)
        dmin = jnp.min(d, axis=1, keepdims=True)
        cand = jnp.where(d == dmin,
                         lax.broadcasted_iota(jnp.int32, d.shape, 1), NE)
        idx = jnp.min(cand, axis=1).astype(jnp.int32)
        onehot = (idx[:, None] ==
                  lax.broadcasted_iota(jnp.int32, d.shape, 1))
        return idx, onehot.astype(jnp.float32)

    def decode(zq):
        hd = jnp.maximum(zq @ Wd0_ref[...] + bd0_ref[...][None, :], 0.0)
        hd = jnp.maximum(hd @ Wd1_ref[...] + bd1_ref[...][None, :], 0.0)
        return hd @ Wd2_ref[...] + bd2_ref[...][None, :]

    @pl.when(i == 0)
    def _init():
        acc_sq[...] = jnp.zeros_like(acc_sq)
        acc_cnt[...] = jnp.zeros_like(acc_cnt)
        acc_ecnt[...] = jnp.zeros_like(acc_ecnt)
        cn_s[...] = _rowsq_xla_row(cb)
        ep = encode(jnp.zeros((1, 3), jnp.float32))
        ip, op = vq(ep, _rowsq_xla_tree(ep))
        zp = op @ cb
        pad_enc[...] = ep
        pad_zq[...] = zp
        pad_rec[...] = decode(zp)
        pad_idx[...] = ip[:, None]

    src = jnp.minimum(start + p0, N)

    @pl.when(p0 < cnt)
    def _full():
        x = coords_ref[pl.ds(src, TB), :]
        rel = p0 + lax.broadcasted_iota(jnp.int32, (TB, 1), 0)
        m = rel < cnt
        mf = m.astype(jnp.float32)
        enc = encode(x * mf)
        enc_out[...] = enc
        zn = jnp.reshape(_rowsq_xla_row(enc), (TB, 1))
        idx, onehot = vq(enc, zn)
        zq = onehot @ cb
        zq_out[...] = zq
        rec_out[...] = decode(zq)
        idx_out[...] = idx[:, None]
        mask_out[...] = m.astype(jnp.int32)
        diff = (zq - enc) * mf
        acc_sq[...] += jnp.sum(diff * diff).reshape(1, 1)
        acc_cnt[...] += jnp.sum(mf).reshape(1, 1)
        acc_ecnt[...] += jnp.sum(onehot * mf, axis=0, keepdims=True)

    @pl.when(p0 >= cnt)
    def _padblk():
        enc_out[...] = jnp.broadcast_to(pad_enc[...], (TB, EDIM))
        zq_out[...] = jnp.broadcast_to(pad_zq[...], (TB, EDIM))
        rec_out[...] = jnp.broadcast_to(pad_rec[...], (TB, 3))
        idx_out[...] = jnp.broadcast_to(pad_idx[...], (TB, 1))
        mask_out[...] = jnp.zeros((TB, 1), jnp.int32)

    @pl.when(i == NBLK - 1)
    def _fin():
        s = acc_sq[...]
        c = acc_cnt[...]
        denom = c * EDIM + 1e-8
        loss_out[...] = BETA * s / denom + s / denom
        e_mean = acc_ecnt[...] / (c + 1e-8)
        ent = jnp.sum(e_mean * jnp.log(e_mean + 1e-10), axis=1, keepdims=True)
        perp_out[...] = jnp.exp(-ent)


def kernel(coords_ca, We0, be0, We1, be1, We2, be2, We3, be3, We4, be4,
           codebook, Wd0, bd0, Wd1, bd1, Wd2, bd2, batch_ids):
    edges = jnp.searchsorted(batch_ids,
                             jnp.arange(B + 1, dtype=jnp.int32),
                             side='left').astype(jnp.int32)
    starts = edges[:B]
    counts = edges[1:] - edges[:B]
    coords_pad = jnp.pad(coords_ca, ((0, TB), (0, 0)))

    full = lambda shape: pl.BlockSpec(shape, lambda i: (0,) * len(shape))
    smem = pl.BlockSpec(memory_space=pltpu.SMEM)

    out_shapes = (
        jax.ShapeDtypeStruct((B * L, EDIM), jnp.float32),
        jax.ShapeDtypeStruct((B * L, EDIM), jnp.float32),
        jax.ShapeDtypeStruct((B * L, 3), jnp.float32),
        jax.ShapeDtypeStruct((B * L, 1), jnp.int32),
        jax.ShapeDtypeStruct((B * L, 1), jnp.int32),
        jax.ShapeDtypeStruct((1, 1), jnp.float32),
        jax.ShapeDtypeStruct((1, 1), jnp.float32),
    )
    out_specs = (
        pl.BlockSpec((TB, EDIM), lambda i: (i, 0)),
        pl.BlockSpec((TB, EDIM), lambda i: (i, 0)),
        pl.BlockSpec((TB, 3), lambda i: (i, 0)),
        pl.BlockSpec((TB, 1), lambda i: (i, 0)),
        pl.BlockSpec((TB, 1), lambda i: (i, 0)),
        pl.BlockSpec((1, 1), lambda i: (0, 0)),
        pl.BlockSpec((1, 1), lambda i: (0, 0)),
    )
    in_specs = [
        smem,
        smem,
        full((N + TB, 3)),
        full((3, H)), full((H,)),
        full((H, H)), full((H,)),
        full((H, H)), full((H,)),
        full((H, H)), full((H,)),
        full((H, EDIM)), full((EDIM,)),
        full((NE, EDIM)),
        full((EDIM, H)), full((H,)),
        full((H, H)), full((H,)),
        full((H, 3)), full((3,)),
    ]

    enc, zq, rec, idx, mask, loss, perp = pl.pallas_call(
        _fused,
        grid=(NBLK,),
        in_specs=in_specs,
        out_specs=out_specs,
        out_shape=out_shapes,
        scratch_shapes=[
            pltpu.VMEM((1, 1), jnp.float32),
            pltpu.VMEM((1, 1), jnp.float32),
            pltpu.VMEM((1, NE), jnp.float32),
            pltpu.VMEM((1, EDIM), jnp.float32),
            pltpu.VMEM((1, EDIM), jnp.float32),
            pltpu.VMEM((1, 3), jnp.float32),
            pltpu.VMEM((1, 1), jnp.int32),
            pltpu.VMEM((1, NE), jnp.float32),
        ],
        compiler_params=pltpu.CompilerParams(
            dimension_semantics=("arbitrary",),
        ),
    )(starts, counts, coords_pad,
      We0, be0, We1, be1, We2, be2, We3, be3, We4, be4, codebook,
      Wd0, bd0, Wd1, bd1, Wd2, bd2)

    reconstructed = rec.reshape(B, L, 3)
    encoded = enc.reshape(B, L, EDIM)
    zq_st = zq.reshape(B, L, EDIM)
    mask_out = mask.reshape(B, L) != 0
    min_idx = idx.reshape(B, L)
    return (reconstructed, loss.reshape(()), perp.reshape(()),
            encoded, zq_st, mask_out, min_idx)


# speedup vs baseline: 1.2877x; 1.2877x over previous
import jax
import jax.numpy as jnp
from jax import lax
from jax.experimental import pallas as pl
from jax.experimental.pallas import tpu as pltpu

B = 8
L = 2048
N = 8192
H = 256
EDIM = 256
NE = 512
BETA = 0.25

TB = 512
HB = 256
PBB = L // TB
NBLK = (B * L) // TB


def _rowsq_xla_tree(x):
    z2 = x * x
    b = z2[:, :128] + z2[:, 128:]
    t = b[:, 0:8]
    for j in range(1, 16):
        t = t + b[:, 8 * j:8 * j + 8]
    u1 = t[:, 0:4] + t[:, 4:8]
    u2 = u1[:, 0:2] + u1[:, 2:4]
    return u2[:, 0:1] + u2[:, 1:2]


def _rowsq_xla_row(x):
    z2 = x * x
    b = z2[:, :128] + z2[:, 128:]
    chunks = []
    for t in range(x.shape[0] // 128):
        bt = b[128 * t:128 * (t + 1), :].T
        s = bt[0:8, :]
        for j in range(1, 16):
            s = s + bt[8 * j:8 * (j + 1), :]
        s = s + pltpu.roll(s, 4, axis=0)
        s = s + pltpu.roll(s, 2, axis=0)
        s = s + pltpu.roll(s, 1, axis=0)
        chunks.append(s[0:1, :])
    return chunks[0] if len(chunks) == 1 else jnp.concatenate(chunks, axis=1)


def _fused(starts_ref, counts_ref, coords_ref,
           We0_ref, be0_ref, We1_ref, be1_ref, We2_ref, be2_ref,
           We3_ref, be3_ref, We4_ref, be4_ref, cb_ref,
           Wd0_ref, bd0_ref, Wd1_ref, bd1_ref, Wd2_ref, bd2_ref,
           enc_out, zq_out, rec_out, idx_out, mask_out, loss_out, perp_out,
           acc_sq, acc_cnt, acc_ecnt, pad_enc, pad_zq, pad_rec, pad_idx,
           cn_s):
    i = pl.program_id(0)
    b = i // PBB
    p0 = (i % PBB) * TB
    start = starts_ref[b]
    cnt = counts_ref[b]
    cb = cb_ref[...]

    def encode(x):
        h = jnp.maximum(x @ We0_ref[...] + be0_ref[...][None, :], 0.0)
        h = jnp.maximum(h @ We1_ref[...] + be1_ref[...][None, :], 0.0)
        h = jnp.maximum(h @ We2_ref[...] + be2_ref[...][None, :], 0.0)
        h = jnp.maximum(h @ We3_ref[...] + be3_ref[...][None, :], 0.0)
        enc = h @ We4_ref[...] + be4_ref[...][None, :]
        return jnp.clip(enc, -10.0, 10.0)

    def vq(enc, zn):
        d = zn + cn_s[...] - 2.0 * (enc @ cb.T)
        dmin = jnp.min(d, axis=1, keepdims=True)
        cand = jnp.where(d == dmin,
                         lax.broadcasted_iota(jnp.int32, d.shape, 1), NE)
        idx = jnp.min(cand, axis=1).astype(jnp.int32)
        onehot = (idx[:, None] ==
                  lax.broadcasted_iota(jnp.int32, d.shape, 1))
        return idx, onehot.astype(jnp.float32)

    def decode(zq):
        hd = jnp.maximum(zq @ Wd0_ref[...] + bd0_ref[...][None, :], 0.0)
        hd = jnp.maximum(hd @ Wd1_ref[...] + bd1_ref[...][None, :], 0.0)
        return hd @ Wd2_ref[...] + bd2_ref[...][None, :]

    @pl.when(i == 0)
    def _init():
        acc_sq[...] = jnp.zeros_like(acc_sq)
        acc_cnt[...] = jnp.zeros_like(acc_cnt)
        acc_ecnt[...] = jnp.zeros_like(acc_ecnt)
        cn_s[...] = _rowsq_xla_row(cb)
        ep = encode(jnp.zeros((1, 3), jnp.float32))
        ip, op = vq(ep, _rowsq_xla_tree(ep))
        zp = op @ cb
        pad_enc[...] = ep
        pad_zq[...] = zp
        pad_rec[...] = decode(zp)
        pad_idx[...] = ip[:, None]

    src = jnp.minimum(start + p0, N)

    @pl.when(p0 < cnt)
    def _full():
        x = coords_ref[pl.ds(src, TB), :]
        rel = p0 + lax.broadcasted_iota(jnp.int32, (TB, 1), 0)
        m = rel < cnt
        mf = m.astype(jnp.float32)
        enc = encode(x * mf)
        enc_out[...] = enc
        zn = jnp.reshape(_rowsq_xla_row(enc), (TB, 1))
        idx, onehot = vq(enc, zn)
        zq = onehot @ cb
        zq_out[...] = zq
        rec_out[...] = decode(zq)
        idx_out[...] = idx[:, None]
        mask_out[...] = m.astype(jnp.int32)
        diff = (zq - enc) * mf
        acc_sq[...] += jnp.sum(diff * diff).reshape(1, 1)
        acc_cnt[...] += jnp.sum(mf).reshape(1, 1)
        acc_ecnt[...] += jnp.sum(onehot * mf, axis=0, keepdims=True)

    @pl.when(p0 >= cnt)
    def _padblk():
        enc_out[...] = jnp.broadcast_to(pad_enc[...], (TB, EDIM))
        zq_out[...] = jnp.broadcast_to(pad_zq[...], (TB, EDIM))
        rec_out[...] = jnp.broadcast_to(pad_rec[...], (TB, 3))
        idx_out[...] = jnp.broadcast_to(pad_idx[...], (TB, 1))
        mask_out[...] = jnp.zeros((TB, 1), jnp.int32)

    @pl.when(i == NBLK - 1)
    def _fin():
        s = acc_sq[...]
        c = acc_cnt[...]
        denom = c * EDIM + 1e-8
        loss_out[...] = BETA * s / denom + s / denom
        e_mean = acc_ecnt[...] / (c + 1e-8)
        ent = jnp.sum(e_mean * jnp.log(e_mean + 1e-10), axis=1, keepdims=True)
        perp_out[...] = jnp.exp(-ent)


def kernel(coords_ca, We0, be0, We1, be1, We2, be2, We3, be3, We4, be4,
           codebook, Wd0, bd0, Wd1, bd1, Wd2, bd2, batch_ids):
    edges = jnp.searchsorted(batch_ids,
                             jnp.arange(B + 1, dtype=jnp.int32),
                             side='left').astype(jnp.int32)
    starts = edges[:B]
    counts = edges[1:] - edges[:B]
    coords_pad = jnp.pad(coords_ca, ((0, TB), (0, 0)))

    full = lambda shape: pl.BlockSpec(shape, lambda i: (0,) * len(shape))
    smem = pl.BlockSpec(memory_space=pltpu.SMEM)

    out_shapes = (
        jax.ShapeDtypeStruct((B * L, EDIM), jnp.float32),
        jax.ShapeDtypeStruct((B * L, EDIM), jnp.float32),
        jax.ShapeDtypeStruct((B * L, 3), jnp.float32),
        jax.ShapeDtypeStruct((B * L, 1), jnp.int32),
        jax.ShapeDtypeStruct((B * L, 1), jnp.int32),
        jax.ShapeDtypeStruct((1, 1), jnp.float32),
        jax.ShapeDtypeStruct((1, 1), jnp.float32),
    )
    out_specs = (
        pl.BlockSpec((TB, EDIM), lambda i: (i, 0)),
        pl.BlockSpec((TB, EDIM), lambda i: (i, 0)),
        pl.BlockSpec((TB, 3), lambda i: (i, 0)),
        pl.BlockSpec((TB, 1), lambda i: (i, 0)),
        pl.BlockSpec((TB, 1), lambda i: (i, 0)),
        pl.BlockSpec((1, 1), lambda i: (0, 0)),
        pl.BlockSpec((1, 1), lambda i: (0, 0)),
    )
    in_specs = [
        smem,
        smem,
        full((N + TB, 3)),
        full((3, H)), full((H,)),
        full((H, H)), full((H,)),
        full((H, H)), full((H,)),
        full((H, H)), full((H,)),
        full((H, EDIM)), full((EDIM,)),
        full((NE, EDIM)),
        full((EDIM, H)), full((H,)),
        full((H, H)), full((H,)),
        full((H, 3)), full((3,)),
    ]

    enc, zq, rec, idx, mask, loss, perp = pl.pallas_call(
        _fused,
        grid=(NBLK,),
        in_specs=in_specs,
        out_specs=out_specs,
        out_shape=out_shapes,
        scratch_shapes=[
            pltpu.VMEM((1, 1), jnp.float32),
            pltpu.VMEM((1, 1), jnp.float32),
            pltpu.VMEM((1, NE), jnp.float32),
            pltpu.VMEM((1, EDIM), jnp.float32),
            pltpu.VMEM((1, EDIM), jnp.float32),
            pltpu.VMEM((1, 3), jnp.float32),
            pltpu.VMEM((1, 1), jnp.int32),
            pltpu.VMEM((1, NE), jnp.float32),
        ],
        compiler_params=pltpu.CompilerParams(
            dimension_semantics=("arbitrary",),
        ),
    )(starts, counts, coords_pad,
      We0, be0, We1, be1, We2, be2, We3, be3, We4, be4, codebook,
      Wd0, bd0, Wd1, bd1, Wd2, bd2)

    reconstructed = rec.reshape(B, L, 3)
    encoded = enc.reshape(B, L, EDIM)
    zq_st = zq.reshape(B, L, EDIM)
    mask_out = mask.reshape(B, L) != 0
    min_idx = idx.reshape(B, L)
    return (reconstructed, loss.reshape(()), perp.reshape(()),
            encoded, zq_st, mask_out, min_idx)
